# Initial kernel scaffold; baseline (speedup 1.0000x reference)
import jax
import jax.numpy as jnp
from jax.experimental import pallas as pl


def kernel(node_feats, smask, smask_full, batch_ids, motif_ids, W_aw, b_aw, W_feat, b_feat, W1, b1, W2, b2):
    raise NotImplementedError("write your pallas kernel here")


# speedup vs baseline: 1.2004x; 1.2004x over previous
import jax
import jax.numpy as jnp
from jax import lax
from jax.experimental import pallas as pl
from jax.experimental.pallas import tpu as pltpu

N = 100000
D = 128
H = 256
B = 1024
M = 4096

NB = 512
NBLK = (N + NB - 1) // NB
NPAD = NBLK * NB


def _pool_body(f_ref, sm_ref, smf_ref, bid_ref, mid_ref, waw_ref, baw_ref,
               graph_ref, hsub_ref):
    step = pl.program_id(0)

    @pl.when(step == 0)
    def _init():
        graph_ref[...] = jnp.zeros_like(graph_ref)
        hsub_ref[...] = jnp.zeros_like(hsub_ref)

    f = f_ref[...]
    waw = waw_ref[...]
    t = jnp.sum(f * waw, axis=1) + baw_ref[0, 0]
    w = jax.nn.sigmoid(t)
    sm = sm_ref[0, 0, :]
    smf = smf_ref[0, 0, :]
    bid = bid_ref[0, 0, :]
    mid = mid_ref[0, 0, :]
    wg = w * sm
    ws = w * smf * (mid > 0).astype(jnp.float32)

    cols_b = lax.broadcasted_iota(jnp.int32, (NB, B), 1)
    oh_b = jnp.where(bid[:, None] == cols_b, wg[:, None], 0.0)
    graph_ref[...] += lax.dot_general(
        oh_b, f, (((0,), (0,)), ((), ())),
        preferred_element_type=jnp.float32)

    cols_m = lax.broadcasted_iota(jnp.int32, (NB, M), 1)
    oh_m = jnp.where((mid[:, None] - 1) == cols_m, ws[:, None], 0.0)
    hsub_ref[...] += lax.dot_general(
        oh_m, f, (((0,), (0,)), ((), ())),
        preferred_element_type=jnp.float32)


def _mlp_body(x_ref, wf_ref, bf_ref, w1_ref, b1_ref, w2_ref, b2_ref, o_ref):
    x = x_ref[...]
    h0 = jnp.dot(x, wf_ref[...], preferred_element_type=jnp.float32) + bf_ref[...]
    h1 = jnp.maximum(
        jnp.dot(h0, w1_ref[...], preferred_element_type=jnp.float32) + b1_ref[...],
        0.0)
    o_ref[...] = jnp.dot(h1, w2_ref[...], preferred_element_type=jnp.float32) + b2_ref[...]


def kernel(node_feats, smask, smask_full, batch_ids, motif_ids,
           W_aw, b_aw, W_feat, b_feat, W1, b1, W2, b2):
    pad = NPAD - N
    f = jnp.pad(node_feats, ((0, pad), (0, 0)))
    sm = jnp.pad(smask, (0, pad)).reshape(NBLK, 1, NB)
    smf = jnp.pad(smask_full, (0, pad)).reshape(NBLK, 1, NB)
    bid = jnp.pad(batch_ids, (0, pad)).reshape(NBLK, 1, NB)
    mid = jnp.pad(motif_ids, (0, pad)).reshape(NBLK, 1, NB)

    graph_feats, h_subs = pl.pallas_call(
        _pool_body,
        grid=(NBLK,),
        in_specs=[
            pl.BlockSpec((NB, D), lambda i: (i, 0)),
            pl.BlockSpec((1, 1, NB), lambda i: (i, 0, 0)),
            pl.BlockSpec((1, 1, NB), lambda i: (i, 0, 0)),
            pl.BlockSpec((1, 1, NB), lambda i: (i, 0, 0)),
            pl.BlockSpec((1, 1, NB), lambda i: (i, 0, 0)),
            pl.BlockSpec((1, D), lambda i: (0, 0)),
            pl.BlockSpec((1, 1), lambda i: (0, 0)),
        ],
        out_specs=[
            pl.BlockSpec((B, D), lambda i: (0, 0)),
            pl.BlockSpec((M, D), lambda i: (0, 0)),
        ],
        out_shape=[
            jax.ShapeDtypeStruct((B, D), jnp.float32),
            jax.ShapeDtypeStruct((M, D), jnp.float32),
        ],
    )(f, sm, smf, bid, mid, W_aw.reshape(1, D), b_aw.reshape(1, 1))

    x = jnp.concatenate([graph_feats, h_subs], axis=0)
    out = pl.pallas_call(
        _mlp_body,
        grid=((B + M) // 512,),
        in_specs=[
            pl.BlockSpec((512, D), lambda i: (i, 0)),
            pl.BlockSpec((D, H), lambda i: (0, 0)),
            pl.BlockSpec((1, H), lambda i: (0, 0)),
            pl.BlockSpec((H, H), lambda i: (0, 0)),
            pl.BlockSpec((1, H), lambda i: (0, 0)),
            pl.BlockSpec((H, H // 2), lambda i: (0, 0)),
            pl.BlockSpec((1, H // 2), lambda i: (0, 0)),
        ],
        out_specs=pl.BlockSpec((512, H // 2), lambda i: (i, 0)),
        out_shape=jax.ShapeDtypeStruct((B + M, H // 2), jnp.float32),
    )(x, W_feat, b_feat.reshape(1, H), W1, b1.reshape(1, H),
      W2, b2.reshape(1, H // 2))

    return (graph_feats, out[:B], out[B:])


# speedup vs baseline: 1.9250x; 1.6036x over previous
import functools

import jax
import jax.numpy as jnp
from jax import lax
from jax.experimental import pallas as pl
from jax.experimental.pallas import tpu as pltpu
from jax.experimental.pallas import tpu_sc as plsc

N = 100000
D = 128
H = 256
B = 1024
M = 4096

NC = 2
NS = 16
NW = NC * NS

CHUNK = 3128
BKN = 224
NFULL = 13
ACC_ROWS = B + M
RPS = ACC_ROWS // NS
NCH = D // 16


def _sc_body(nf, sm, smf, bid, mid, waw, baw, zrows, parts,
             acc, fbuf, smb, smfb, bidb, midb, wawb, bawb,
             stage_g, stage_m, idxg, idxm):
    core = lax.axis_index("c")
    sid = lax.axis_index("s")
    wid = core * NS + sid

    pltpu.sync_copy(zrows.at[pl.ds(sid * RPS * D, RPS * D)],
                    acc.at[pl.ds(sid * RPS * D, RPS * D)])
    plsc.subcore_barrier()

    base = wid * CHUNK
    last = wid == NW - 1
    tail_off = jnp.where(last, 2808, 2904)
    tail_lo = jnp.where(last, 104, 8)

    pltpu.sync_copy(waw, wawb)
    pltpu.sync_copy(baw, bawb)
    wawc = [wawb[pl.ds(c * 16, 16)] for c in range(NCH)]
    bval = bawb[pl.ds(0, 16)][0]
    iota = lax.iota(jnp.int32, 16)
    z16 = jnp.zeros((16,), jnp.int32)
    zv = jnp.zeros((16,), jnp.float32)
    lane0 = iota == 0

    def flush(stage, idx, row, accs):
        rb = row * D
        for c in range(NCH):
            stage[pl.ds(c * 16, 16)] = accs[c]
            idx[pl.ds(c * 16, 16)] = rb + c * 16 + iota
        pltpu.sync_copy(stage, acc.at[idx], add=True)

    def node_step(i, carry, lo):
        (cur_b, cur_m), ag, am = carry
        bid_i = bidb[pl.ds(i, 16)][0]
        mid_i = midb[pl.ds(i, 16)][0]
        chg_b = bid_i != cur_b
        chg_m = mid_i != cur_m

        @pl.when(chg_b)
        def _():
            flush(stage_g, idxg, cur_b, ag)

        @pl.when(chg_m)
        def _():
            flush(stage_m, idxm, B - 1 + cur_m, am)

        fc = [fbuf[pl.ds(i * D + c * 16, 16)] for c in range(NCH)]
        dv = fc[0] * wawc[0]
        for c in range(1, NCH):
            dv = dv + fc[c] * wawc[c]
        for s in (8, 4, 2, 1):
            dv = dv + dv.at[iota ^ s].get(mode="promise_in_bounds")
        wv = dv + bval
        wv = 1.0 / (1.0 + jnp.exp(-wv))
        validf = jnp.where(i >= lo, 1.0, 0.0)
        keepf = jnp.where(mid_i > 0, 1.0, 0.0)
        sm_i = smb[pl.ds(i, 16)][0]
        smf_i = smfb[pl.ds(i, 16)][0]
        wgv = wv * (sm_i * validf)
        wsv = wv * (smf_i * validf * keepf)
        ag = [jnp.where(chg_b, zv, a) + f * wgv for a, f in zip(ag, fc)]
        am = [jnp.where(chg_m, zv, a) + f * wsv for a, f in zip(am, fc)]
        return (bid_i, mid_i), ag, am

    def block_step(b, carry):
        boff = lax.select(b == NFULL, tail_off, b * BKN)
        lo = lax.select(b == NFULL, tail_lo, 0)
        off = base + boff
        pltpu.sync_copy(nf.at[pl.ds(off * D, BKN * D)], fbuf)
        pltpu.sync_copy(sm.at[pl.ds(off, BKN)], smb.at[pl.ds(0, BKN)])
        pltpu.sync_copy(smf.at[pl.ds(off, BKN)], smfb.at[pl.ds(0, BKN)])
        pltpu.sync_copy(bid.at[pl.ds(off, BKN)], bidb.at[pl.ds(0, BKN)])
        pltpu.sync_copy(mid.at[pl.ds(off, BKN)], midb.at[pl.ds(0, BKN)])
        return lax.fori_loop(
            0, BKN, lambda i, c: node_step(i, c, lo), carry)

    carry0 = ((jnp.int32(0), jnp.int32(0)),
              [zv] * NCH, [zv] * NCH)
    (cur_b, cur_m), ag, am = lax.fori_loop(0, NFULL + 1, block_step, carry0)
    flush(stage_g, idxg, cur_b, ag)
    flush(stage_m, idxm, B - 1 + cur_m, am)

    plsc.subcore_barrier()
    pltpu.sync_copy(acc.at[pl.ds(sid * RPS * D, RPS * D)],
                    parts.at[core, pl.ds(sid * RPS * D, RPS * D)])


def _sc_pool(nf_flat, sm, smf, bid, mid, waw, baw16, zrows):
    return pl.kernel(
        _sc_body,
        out_type=jax.ShapeDtypeStruct((NC, ACC_ROWS * D), jnp.float32),
        mesh=plsc.VectorSubcoreMesh(core_axis_name="c", subcore_axis_name="s"),
        scratch_types=[
            pltpu.VMEM_SHARED((ACC_ROWS * D,), jnp.float32),
            pltpu.VMEM((BKN * D,), jnp.float32),
            pltpu.VMEM((BKN + 16,), jnp.float32),
            pltpu.VMEM((BKN + 16,), jnp.float32),
            pltpu.VMEM((BKN + 16,), jnp.int32),
            pltpu.VMEM((BKN + 16,), jnp.int32),
            pltpu.VMEM((D,), jnp.float32),
            pltpu.VMEM((16,), jnp.float32),
            pltpu.VMEM((D,), jnp.float32),
            pltpu.VMEM((D,), jnp.float32),
            pltpu.VMEM((D,), jnp.int32),
            pltpu.VMEM((D,), jnp.int32),
        ],
    )(nf_flat, sm, smf, bid, mid, waw, baw16, zrows)


def _mlp_body(p_ref, wf_ref, bf_ref, w1_ref, b1_ref, w2_ref, b2_ref,
              x_ref, o_ref):
    x = p_ref[0] + p_ref[1]
    x_ref[...] = x
    h0 = jnp.dot(x, wf_ref[...], preferred_element_type=jnp.float32) + bf_ref[...]
    h1 = jnp.maximum(
        jnp.dot(h0, w1_ref[...], preferred_element_type=jnp.float32) + b1_ref[...],
        0.0)
    o_ref[...] = jnp.dot(h1, w2_ref[...], preferred_element_type=jnp.float32) + b2_ref[...]


def kernel(node_feats, smask, smask_full, batch_ids, motif_ids,
           W_aw, b_aw, W_feat, b_feat, W1, b1, W2, b2):
    nf_flat = node_feats.reshape(-1)
    zrows = jnp.zeros((ACC_ROWS * D,), jnp.float32)
    parts = _sc_pool(nf_flat, smask, smask_full, batch_ids, motif_ids,
                     W_aw.reshape(D), jnp.pad(b_aw, (0, 15)), zrows)
    parts = parts.reshape(NC, ACC_ROWS, D)

    xsum, out = pl.pallas_call(
        _mlp_body,
        grid=(ACC_ROWS // 512,),
        in_specs=[
            pl.BlockSpec((NC, 512, D), lambda i: (0, i, 0)),
            pl.BlockSpec((D, H), lambda i: (0, 0)),
            pl.BlockSpec((1, H), lambda i: (0, 0)),
            pl.BlockSpec((H, H), lambda i: (0, 0)),
            pl.BlockSpec((1, H), lambda i: (0, 0)),
            pl.BlockSpec((H, H // 2), lambda i: (0, 0)),
            pl.BlockSpec((1, H // 2), lambda i: (0, 0)),
        ],
        out_specs=[
            pl.BlockSpec((512, D), lambda i: (i, 0)),
            pl.BlockSpec((512, H // 2), lambda i: (i, 0)),
        ],
        out_shape=[
            jax.ShapeDtypeStruct((ACC_ROWS, D), jnp.float32),
            jax.ShapeDtypeStruct((ACC_ROWS, H // 2), jnp.float32),
        ],
    )(parts, W_feat, b_feat.reshape(1, H), W1, b1.reshape(1, H),
      W2, b2.reshape(1, H // 2))

    return (xsum[:B], out[:B], out[B:])
